# baseline (speedup 1.0000x reference)
import jax
import jax.numpy as jnp
from jax.experimental import pallas as pl

_MAX_ITER = 3
_IGNORE_INDEX = -100


def _body(mask_ref, out_ref):
    out_ref[...] = jnp.where(
        mask_ref[...] == 1, jnp.int32(_MAX_ITER), jnp.int32(_IGNORE_INDEX)
    )


def kernel(active_logits, active_labels_shifted, iter_depth,
           current_iter_mask, active_valid_mask):
    return pl.pallas_call(
        _body,
        out_shape=jax.ShapeDtypeStruct(active_valid_mask.shape, jnp.int32),
    )(active_valid_mask)
